# baseline (speedup 1.0000x reference)
import functools

import jax
import jax.numpy as jnp
from jax import lax
from jax.experimental import pallas as pl
from jax.experimental.pallas import tpu as pltpu
from jax.experimental.pallas import tpu_sc as plsc

N = 10000
E = 320000
D = 128
H = 16
C = 16

NC = 2
NS = 16
NW = NC * NS
CHUNK = 500
NCH = 20
NBUF = 4
ACC_ROWS = 10240
RPT = ACC_ROWS // NS

_mesh = plsc.VectorSubcoreMesh(core_axis_name="c", subcore_axis_name="s")
_sc_params = pltpu.CompilerParams(use_tc_tiling_on_sc=False)


def _fill_rows(ref, nrows, value):
    vec = jnp.full((16,), value, jnp.float32)

    def body(i, carry):
        ref[i, :] = vec
        return carry

    lax.fori_loop(0, nrows, body, 0)


@functools.partial(
    pl.kernel,
    out_type=jax.ShapeDtypeStruct((NC, ACC_ROWS, H), jnp.float32),
    mesh=_mesh,
    scratch_types=[
        pltpu.VMEM((NCH, CHUNK), jnp.int32),
        pltpu.VMEM((CHUNK, H), jnp.float32),
        pltpu.VMEM((RPT, H), jnp.float32),
        pltpu.VMEM_SHARED((ACC_ROWS, H), jnp.float32),
        [pltpu.SemaphoreType.DMA] * NBUF,
    ],
    compiler_params=_sc_params,
)
def _deg_kernel(ei_hbm, out_hbm, dst_v, ones_v, zero_v, acc_sh, sems):
    cid = lax.axis_index("c")
    sid = lax.axis_index("s")
    wid = sid * NC + cid
    cp = pltpu.async_copy(ei_hbm.at[1, wid], dst_v, sems[0])
    _fill_rows(ones_v, CHUNK, 1.0)
    _fill_rows(zero_v, RPT, 0.0)
    pltpu.sync_copy(zero_v, acc_sh.at[pl.ds(sid * RPT, RPT)])
    cp.wait()
    plsc.subcore_barrier()

    for b in range(NBUF):
        pltpu.async_copy(ones_v, acc_sh.at[dst_v.at[b]], sems[b], add=True)

    def body(i, carry):
        for b in range(NBUF):
            j = NBUF * i + b
            pltpu.make_async_copy(ones_v, acc_sh.at[dst_v.at[j]],
                                  sems[b]).wait()
            pltpu.async_copy(ones_v, acc_sh.at[dst_v.at[j + NBUF]], sems[b],
                             add=True)
        return carry

    lax.fori_loop(0, NCH // NBUF - 1, body, 0)
    for b in range(NBUF):
        j = NCH - NBUF + b
        pltpu.make_async_copy(ones_v, acc_sh.at[dst_v.at[j]], sems[b]).wait()
    plsc.subcore_barrier()
    pltpu.sync_copy(acc_sh.at[pl.ds(sid * RPT, RPT)],
                    out_hbm.at[cid, pl.ds(sid * RPT, RPT)])


@functools.partial(
    pl.kernel,
    out_type=jax.ShapeDtypeStruct((NC, ACC_ROWS, H), jnp.float32),
    mesh=_mesh,
    scratch_types=[
        pltpu.VMEM((NCH, CHUNK), jnp.int32),
        pltpu.VMEM((NCH, CHUNK), jnp.int32),
        pltpu.VMEM((RPT, H), jnp.float32),
        [pltpu.VMEM((CHUNK, H), jnp.float32)] * NBUF,
        pltpu.VMEM_SHARED((ACC_ROWS, H), jnp.float32),
        [pltpu.SemaphoreType.DMA] * NBUF,
        [pltpu.SemaphoreType.DMA] * NBUF,
    ],
    compiler_params=_sc_params,
)
def _agg_kernel(g_hbm, ei_hbm, out_hbm,
                src_v, dst_v, zero_v, bufs, acc_sh, gsems, ssems):
    cid = lax.axis_index("c")
    sid = lax.axis_index("s")
    wid = sid * NC + cid
    cps = pltpu.async_copy(ei_hbm.at[0, wid], src_v, gsems[0])
    cpd = pltpu.async_copy(ei_hbm.at[1, wid], dst_v, gsems[1])
    _fill_rows(zero_v, RPT, 0.0)
    pltpu.sync_copy(zero_v, acc_sh.at[pl.ds(sid * RPT, RPT)])
    cps.wait()
    cpd.wait()
    plsc.subcore_barrier()

    for b in range(NBUF):
        pltpu.async_copy(g_hbm.at[src_v.at[b]], bufs[b], gsems[b])

    def body(i, carry):
        for b in range(NBUF):
            j = NBUF * i + b
            pltpu.make_async_copy(g_hbm.at[src_v.at[j]], bufs[b],
                                  gsems[b]).wait()
            pltpu.async_copy(bufs[b], acc_sh.at[dst_v.at[j]], ssems[b],
                             add=True)
        for b in range(NBUF):
            j = NBUF * i + b
            pltpu.make_async_copy(bufs[b], acc_sh.at[dst_v.at[j]],
                                  ssems[b]).wait()
            pltpu.async_copy(g_hbm.at[src_v.at[j + NBUF]], bufs[b], gsems[b])
        return carry

    lax.fori_loop(0, NCH // NBUF - 1, body, 0)
    for b in range(NBUF):
        j = NCH - NBUF + b
        pltpu.make_async_copy(g_hbm.at[src_v.at[j]], bufs[b], gsems[b]).wait()
        pltpu.async_copy(bufs[b], acc_sh.at[dst_v.at[j]], ssems[b], add=True)
    for b in range(NBUF):
        j = NCH - NBUF + b
        pltpu.make_async_copy(bufs[b], acc_sh.at[dst_v.at[j]], ssems[b]).wait()
    plsc.subcore_barrier()
    pltpu.sync_copy(acc_sh.at[pl.ds(sid * RPT, RPT)],
                    out_hbm.at[cid, pl.ds(sid * RPT, RPT)])


def _matmul1_body(x_ref, w1_ref, h_ref):
    h_ref[...] = jnp.dot(x_ref[...], w1_ref[...],
                         preferred_element_type=jnp.float32)


def _scale1_body(h_ref, dp_ref, g_ref, dis_ref):
    dp = dp_ref[...]
    deg = dp[0, :N] + dp[1, :N] + 1.0
    dis = lax.rsqrt(deg)
    g_ref[...] = h_ref[...] * dis
    dis_ref[...] = dis


def _dense2_body(g_ref, ap_ref, dis_ref, b1_ref, w2_ref, out_ref):
    dis = dis_ref[...]
    ap = ap_ref[...]
    z = jnp.maximum(dis * (g_ref[...] + ap[0, :N] + ap[1, :N]) + b1_ref[...],
                    0.0)
    out_ref[...] = jnp.dot(z, w2_ref[...], preferred_element_type=jnp.float32) * dis


def _dense3_body(g_ref, ap_ref, dis_ref, b2_ref, out_ref):
    ap = ap_ref[...]
    out_ref[...] = (dis_ref[...] * (g_ref[...] + ap[0, :N] + ap[1, :N])
                    + b2_ref[...])


def kernel(x, edge_index, W1, b1, W2, b2):
    ei4 = edge_index.reshape(2, NW, NCH, CHUNK)

    degp = _deg_kernel(ei4)

    h1 = pl.pallas_call(
        _matmul1_body,
        out_shape=jax.ShapeDtypeStruct((N, H), jnp.float32),
    )(x, W1)
    g1, dis = pl.pallas_call(
        _scale1_body,
        out_shape=(jax.ShapeDtypeStruct((N, H), jnp.float32),
                   jax.ShapeDtypeStruct((N, H), jnp.float32)),
    )(h1, degp)

    a1p = _agg_kernel(g1, ei4)
    g2 = pl.pallas_call(
        _dense2_body,
        out_shape=jax.ShapeDtypeStruct((N, H), jnp.float32),
    )(g1, a1p, dis, b1.reshape(1, H), W2)

    a2p = _agg_kernel(g2, ei4)
    out = pl.pallas_call(
        _dense3_body,
        out_shape=jax.ShapeDtypeStruct((N, C), jnp.float32),
    )(g2, a2p, dis, b2.reshape(1, C))
    return out


# speedup vs baseline: 1.2386x; 1.2386x over previous
import functools

import jax
import jax.numpy as jnp
from jax import lax
from jax.experimental import pallas as pl
from jax.experimental.pallas import tpu as pltpu
from jax.experimental.pallas import tpu_sc as plsc

N = 10000
E = 320000
D = 128
H = 16
C = 16

NC = 2
NS = 16
NW = NC * NS
CHUNK = 500
NCH = 20
NBUF = 4
SPT = N // NS
APAD = 10048

PR = N * H // 128
PRC = APAD * H // 128

_mesh = plsc.VectorSubcoreMesh(core_axis_name="c", subcore_axis_name="s")
_sc_params = pltpu.CompilerParams(use_tc_tiling_on_sc=False,
                                  needs_layout_passes=False)


def _fill_rows(ref, nrows, value):
    vec = jnp.full((16,), value, jnp.float32)

    def body(i, carry):
        ref[i, :] = vec
        return carry

    lax.fori_loop(0, nrows, body, 0)


def _rsqrt16(p):
    i = plsc.bitcast(p, jnp.int32)
    i = 0x5F3759DF - lax.shift_right_logical(i, 1)
    y = plsc.bitcast(i, jnp.float32)
    for _ in range(3):
        y = y * (1.5 - 0.5 * p * y * y)
    return y


@functools.partial(
    pl.kernel,
    out_type=jax.ShapeDtypeStruct((NC, NS, SPT), jnp.float32),
    mesh=_mesh,
    scratch_types=[
        pltpu.VMEM((NCH, CHUNK), jnp.int32),
        pltpu.VMEM((CHUNK, H), jnp.float32),
        pltpu.VMEM((640, H), jnp.float32),
        pltpu.VMEM((640,), jnp.float32),
        pltpu.VMEM_SHARED((N, H), jnp.float32),
        [pltpu.SemaphoreType.DMA] * NBUF,
    ],
    compiler_params=_sc_params,
)
def _deg_kernel(dst4_hbm, out_hbm, dst_v, ones_v, acc_v, deg1_v, acc_sh, sems):
    cid = lax.axis_index("c")
    sid = lax.axis_index("s")
    wid = sid * NC + cid
    cp = pltpu.async_copy(dst4_hbm.at[wid], dst_v, sems[0])
    _fill_rows(ones_v, CHUNK, 1.0)
    _fill_rows(acc_v, SPT, 0.0)
    pltpu.sync_copy(acc_v.at[pl.ds(0, SPT)], acc_sh.at[pl.ds(sid * SPT, SPT)])
    cp.wait()
    plsc.subcore_barrier()

    for b in range(NBUF):
        pltpu.async_copy(ones_v, acc_sh.at[dst_v.at[b]], sems[b], add=True)

    def body(i, carry):
        for b in range(NBUF):
            j = NBUF * i + b
            pltpu.make_async_copy(ones_v, acc_sh.at[dst_v.at[j]],
                                  sems[b]).wait()
            pltpu.async_copy(ones_v, acc_sh.at[dst_v.at[j + NBUF]], sems[b],
                             add=True)
        return carry

    lax.fori_loop(0, NCH // NBUF - 1, body, 0)
    for b in range(NBUF):
        j = NCH - NBUF + b
        pltpu.make_async_copy(ones_v, acc_sh.at[dst_v.at[j]], sems[b]).wait()
    plsc.subcore_barrier()
    pltpu.sync_copy(acc_sh.at[pl.ds(sid * SPT, SPT)], acc_v.at[pl.ds(0, SPT)])
    zeros16 = jnp.zeros((16,), jnp.int32)

    def extract(k, carry):
        rows = k * 16 + lax.iota(jnp.int32, 16)
        deg1_v[pl.ds(k * 16, 16)] = plsc.load_gather(acc_v, [rows, zeros16])
        return carry

    lax.fori_loop(0, 640 // 16, extract, 0)
    pltpu.sync_copy(deg1_v.at[pl.ds(0, SPT)], out_hbm.at[cid, sid])


@functools.partial(
    pl.kernel,
    out_type=(jax.ShapeDtypeStruct((NC, APAD, H), jnp.float32),
              jax.ShapeDtypeStruct((N, H), jnp.float32),
              jax.ShapeDtypeStruct((N, H), jnp.float32)),
    mesh=_mesh,
    scratch_types=[
        pltpu.VMEM((NCH, CHUNK), jnp.int32),
        pltpu.VMEM((NCH, CHUNK), jnp.int32),
        pltpu.VMEM((SPT, H), jnp.float32),
        pltpu.VMEM((SPT,), jnp.float32),
        pltpu.VMEM((SPT,), jnp.float32),
        pltpu.VMEM((SPT, H), jnp.float32),
        pltpu.VMEM((SPT, H), jnp.float32),
        pltpu.VMEM((SPT, H), jnp.float32),
        [pltpu.VMEM((CHUNK, H), jnp.float32)] * NBUF,
        pltpu.VMEM_SHARED((N, H), jnp.float32),
        pltpu.VMEM_SHARED((N, H), jnp.float32),
        [pltpu.SemaphoreType.DMA] * NBUF,
        [pltpu.SemaphoreType.DMA] * NBUF,
    ],
    compiler_params=_sc_params,
)
def _scale_agg_kernel(h_hbm, degp_hbm, src4_hbm, dst4_hbm,
                      ap_out, g_out, dis_out,
                      src_v, dst_v, h_v, p0_v, p1_v, g_v, dis_v, zero_v,
                      bufs, g_sh, acc_sh, gsems, ssems):
    cid = lax.axis_index("c")
    sid = lax.axis_index("s")
    wid = sid * NC + cid
    cps = pltpu.async_copy(src4_hbm.at[wid], src_v, gsems[0])
    cpd = pltpu.async_copy(dst4_hbm.at[wid], dst_v, gsems[1])
    cph = pltpu.async_copy(h_hbm.at[pl.ds(sid * SPT, SPT)], h_v, gsems[2])
    cp0 = pltpu.async_copy(degp_hbm.at[0, sid], p0_v, gsems[3])
    cp1 = pltpu.async_copy(degp_hbm.at[1, sid], p1_v, ssems[0])
    _fill_rows(zero_v, SPT, 0.0)
    pltpu.sync_copy(zero_v, acc_sh.at[pl.ds(sid * SPT, SPT)])
    cph.wait()
    cp0.wait()
    cp1.wait()

    def srow(i, carry):
        iv = jnp.full((16,), i, jnp.int32)
        p = (plsc.load_gather(p0_v, [iv]) + plsc.load_gather(p1_v, [iv])
             + 1.0)
        y = _rsqrt16(p)
        dis_v[i, :] = y
        g_v[i, :] = h_v[i, :] * y
        return carry

    lax.fori_loop(0, SPT, srow, 0, unroll=4)
    pltpu.sync_copy(g_v, g_sh.at[pl.ds(sid * SPT, SPT)])

    @pl.when(cid == 0)
    def _():
        pltpu.sync_copy(g_v, g_out.at[pl.ds(sid * SPT, SPT)])
        pltpu.sync_copy(dis_v, dis_out.at[pl.ds(sid * SPT, SPT)])

    cps.wait()
    cpd.wait()
    plsc.subcore_barrier()

    for b in range(NBUF):
        pltpu.async_copy(g_sh.at[src_v.at[b]], bufs[b], gsems[b])

    def body(i, carry):
        for b in range(NBUF):
            j = NBUF * i + b
            pltpu.make_async_copy(g_sh.at[src_v.at[j]], bufs[b],
                                  gsems[b]).wait()
            pltpu.async_copy(bufs[b], acc_sh.at[dst_v.at[j]], ssems[b],
                             add=True)
        for b in range(NBUF):
            j = NBUF * i + b
            pltpu.make_async_copy(bufs[b], acc_sh.at[dst_v.at[j]],
                                  ssems[b]).wait()
            pltpu.async_copy(g_sh.at[src_v.at[j + NBUF]], bufs[b], gsems[b])
        return carry

    lax.fori_loop(0, NCH // NBUF - 1, body, 0)
    for b in range(NBUF):
        j = NCH - NBUF + b
        pltpu.make_async_copy(g_sh.at[src_v.at[j]], bufs[b], gsems[b]).wait()
        pltpu.async_copy(bufs[b], acc_sh.at[dst_v.at[j]], ssems[b], add=True)
    for b in range(NBUF):
        j = NCH - NBUF + b
        pltpu.make_async_copy(bufs[b], acc_sh.at[dst_v.at[j]], ssems[b]).wait()
    plsc.subcore_barrier()
    pltpu.sync_copy(acc_sh.at[pl.ds(sid * SPT, SPT)],
                    ap_out.at[cid, pl.ds(sid * SPT, SPT)])


@functools.partial(
    pl.kernel,
    out_type=jax.ShapeDtypeStruct((NC, APAD, H), jnp.float32),
    mesh=_mesh,
    scratch_types=[
        pltpu.VMEM((NCH, CHUNK), jnp.int32),
        pltpu.VMEM((NCH, CHUNK), jnp.int32),
        pltpu.VMEM((SPT, H), jnp.float32),
        [pltpu.VMEM((CHUNK, H), jnp.float32)] * NBUF,
        pltpu.VMEM_SHARED((N, H), jnp.float32),
        [pltpu.SemaphoreType.DMA] * NBUF,
        [pltpu.SemaphoreType.DMA] * NBUF,
    ],
    compiler_params=_sc_params,
)
def _agg_kernel(g_hbm, src4_hbm, dst4_hbm, out_hbm,
                src_v, dst_v, zero_v, bufs, acc_sh, gsems, ssems):
    cid = lax.axis_index("c")
    sid = lax.axis_index("s")
    wid = sid * NC + cid
    cps = pltpu.async_copy(src4_hbm.at[wid], src_v, gsems[0])
    cpd = pltpu.async_copy(dst4_hbm.at[wid], dst_v, gsems[1])
    _fill_rows(zero_v, SPT, 0.0)
    pltpu.sync_copy(zero_v, acc_sh.at[pl.ds(sid * SPT, SPT)])
    cps.wait()
    cpd.wait()
    plsc.subcore_barrier()

    for b in range(NBUF):
        pltpu.async_copy(g_hbm.at[src_v.at[b]], bufs[b], gsems[b])

    def body(i, carry):
        for b in range(NBUF):
            j = NBUF * i + b
            pltpu.make_async_copy(g_hbm.at[src_v.at[j]], bufs[b],
                                  gsems[b]).wait()
            pltpu.async_copy(bufs[b], acc_sh.at[dst_v.at[j]], ssems[b],
                             add=True)
        for b in range(NBUF):
            j = NBUF * i + b
            pltpu.make_async_copy(bufs[b], acc_sh.at[dst_v.at[j]],
                                  ssems[b]).wait()
            pltpu.async_copy(g_hbm.at[src_v.at[j + NBUF]], bufs[b], gsems[b])
        return carry

    lax.fori_loop(0, NCH // NBUF - 1, body, 0)
    for b in range(NBUF):
        j = NCH - NBUF + b
        pltpu.make_async_copy(g_hbm.at[src_v.at[j]], bufs[b], gsems[b]).wait()
        pltpu.async_copy(bufs[b], acc_sh.at[dst_v.at[j]], ssems[b], add=True)
    for b in range(NBUF):
        j = NCH - NBUF + b
        pltpu.make_async_copy(bufs[b], acc_sh.at[dst_v.at[j]], ssems[b]).wait()
    plsc.subcore_barrier()
    pltpu.sync_copy(acc_sh.at[pl.ds(sid * SPT, SPT)],
                    out_hbm.at[cid, pl.ds(sid * SPT, SPT)])


def _matmul1_body(x_ref, w1_ref, h_ref):
    h_ref[...] = jnp.dot(x_ref[...], w1_ref[...],
                         preferred_element_type=jnp.float32)


def _dense2_body(gp_ref, ap_ref, disp_ref, b1t_ref, w2b_ref, outp_ref):
    dis = disp_ref[...]
    ap = ap_ref[...]
    z = jnp.maximum(
        dis * (gp_ref[...] + ap[0:PR] + ap[PRC:PRC + PR]) + b1t_ref[...], 0.0)
    outp_ref[...] = jnp.dot(z, w2b_ref[...],
                            preferred_element_type=jnp.float32) * dis


def _dense3_body(gp_ref, ap_ref, disp_ref, b2t_ref, outp_ref):
    ap = ap_ref[...]
    outp_ref[...] = (disp_ref[...] * (gp_ref[...] + ap[0:PR] + ap[PRC:PRC + PR])
                     + b2t_ref[...])


def kernel(x, edge_index, W1, b1, W2, b2):
    src4 = edge_index[0].reshape(NW, NCH, CHUNK)
    dst4 = edge_index[1].reshape(NW, NCH, CHUNK)
    w2b = jnp.kron(jnp.eye(8, dtype=jnp.float32), W2)
    b1t = jnp.tile(b1, 8).reshape(1, 128)
    b2t = jnp.tile(b2, 8).reshape(1, 128)
    pstruct = jax.ShapeDtypeStruct((PR, 128), jnp.float32)

    degp = _deg_kernel(dst4)

    h1 = pl.pallas_call(
        _matmul1_body,
        out_shape=jax.ShapeDtypeStruct((N, H), jnp.float32),
    )(x, W1)

    a1p, g1, dis = _scale_agg_kernel(h1, degp, src4, dst4)
    g1p = g1.reshape(PR, 128)
    disp = dis.reshape(PR, 128)
    g2p = pl.pallas_call(
        _dense2_body, out_shape=pstruct)(
            g1p, a1p.reshape(NC * PRC, 128), disp, b1t, w2b)

    a2p = _agg_kernel(g2p.reshape(N, H), src4, dst4)
    outp = pl.pallas_call(
        _dense3_body, out_shape=pstruct)(
            g2p, a2p.reshape(NC * PRC, 128), disp, b2t)
    return outp.reshape(N, C)


# speedup vs baseline: 1.2388x; 1.0002x over previous
import functools

import jax
import jax.numpy as jnp
from jax import lax
from jax.experimental import pallas as pl
from jax.experimental.pallas import tpu as pltpu
from jax.experimental.pallas import tpu_sc as plsc

N = 10000
E = 320000
D = 128
H = 16
C = 16

NC = 2
NS = 16
NW = NC * NS
CHUNK = 500
NCH = 20
NBUF = 4
SPT = N // NS
APAD = 10048

PR = N * H // 128
PRC = APAD * H // 128

_mesh = plsc.VectorSubcoreMesh(core_axis_name="c", subcore_axis_name="s")
_sc_params = pltpu.CompilerParams(use_tc_tiling_on_sc=False,
                                  needs_layout_passes=False)


def _fill_rows(ref, nrows, value):
    vec = jnp.full((16,), value, jnp.float32)

    def body(i, carry):
        ref[i, :] = vec
        return carry

    lax.fori_loop(0, nrows, body, 0)


def _rsqrt16(p):
    i = plsc.bitcast(p, jnp.int32)
    i = 0x5F3759DF - lax.shift_right_logical(i, 1)
    y = plsc.bitcast(i, jnp.float32)
    for _ in range(3):
        y = y * (1.5 - 0.5 * p * y * y)
    return y


@functools.partial(
    pl.kernel,
    out_type=jax.ShapeDtypeStruct((NC, NS, SPT), jnp.float32),
    mesh=_mesh,
    scratch_types=[
        pltpu.VMEM((NCH, CHUNK), jnp.int32),
        pltpu.VMEM((CHUNK, H), jnp.float32),
        pltpu.VMEM((640, H), jnp.float32),
        pltpu.VMEM((640,), jnp.float32),
        pltpu.VMEM_SHARED((N, H), jnp.float32),
        [pltpu.SemaphoreType.DMA] * NBUF,
    ],
    compiler_params=_sc_params,
)
def _deg_kernel(dst4_hbm, out_hbm, dst_v, ones_v, acc_v, deg1_v, acc_sh, sems):
    cid = lax.axis_index("c")
    sid = lax.axis_index("s")
    wid = sid * NC + cid
    cp = pltpu.async_copy(dst4_hbm.at[wid], dst_v, sems[0])
    _fill_rows(ones_v, CHUNK, 1.0)
    _fill_rows(acc_v, SPT, 0.0)
    pltpu.sync_copy(acc_v.at[pl.ds(0, SPT)], acc_sh.at[pl.ds(sid * SPT, SPT)])
    cp.wait()
    plsc.subcore_barrier()

    for b in range(NBUF):
        pltpu.async_copy(ones_v, acc_sh.at[dst_v.at[b]], sems[b], add=True)

    def body(i, carry):
        for b in range(NBUF):
            j = NBUF * i + b
            pltpu.make_async_copy(ones_v, acc_sh.at[dst_v.at[j]],
                                  sems[b]).wait()
            pltpu.async_copy(ones_v, acc_sh.at[dst_v.at[j + NBUF]], sems[b],
                             add=True)
        return carry

    lax.fori_loop(0, NCH // NBUF - 1, body, 0)
    for b in range(NBUF):
        j = NCH - NBUF + b
        pltpu.make_async_copy(ones_v, acc_sh.at[dst_v.at[j]], sems[b]).wait()
    plsc.subcore_barrier()
    pltpu.sync_copy(acc_sh.at[pl.ds(sid * SPT, SPT)], acc_v.at[pl.ds(0, SPT)])
    zeros16 = jnp.zeros((16,), jnp.int32)

    def extract(k, carry):
        rows = k * 16 + lax.iota(jnp.int32, 16)
        deg1_v[pl.ds(k * 16, 16)] = plsc.load_gather(acc_v, [rows, zeros16])
        return carry

    lax.fori_loop(0, 640 // 16, extract, 0)
    pltpu.sync_copy(deg1_v.at[pl.ds(0, SPT)], out_hbm.at[cid, sid])


@functools.partial(
    pl.kernel,
    out_type=(jax.ShapeDtypeStruct((NC, APAD, H), jnp.float32),
              jax.ShapeDtypeStruct((N, H), jnp.float32),
              jax.ShapeDtypeStruct((N, H), jnp.float32)),
    mesh=_mesh,
    scratch_types=[
        pltpu.VMEM((NCH, CHUNK), jnp.int32),
        pltpu.VMEM((NCH, CHUNK), jnp.int32),
        pltpu.VMEM((SPT, H), jnp.float32),
        pltpu.VMEM((SPT,), jnp.float32),
        pltpu.VMEM((SPT,), jnp.float32),
        pltpu.VMEM((SPT, H), jnp.float32),
        pltpu.VMEM((SPT, H), jnp.float32),
        pltpu.VMEM((SPT, H), jnp.float32),
        [pltpu.VMEM((CHUNK, H), jnp.float32)] * NBUF,
        pltpu.VMEM_SHARED((N, H), jnp.float32),
        pltpu.VMEM_SHARED((N, H), jnp.float32),
        [pltpu.SemaphoreType.DMA] * NBUF,
        [pltpu.SemaphoreType.DMA] * NBUF,
    ],
    compiler_params=_sc_params,
)
def _scale_agg_kernel(h_hbm, degp_hbm, src4_hbm, dst4_hbm,
                      ap_out, g_out, dis_out,
                      src_v, dst_v, h_v, p0_v, p1_v, g_v, dis_v, zero_v,
                      bufs, g_sh, acc_sh, gsems, ssems):
    cid = lax.axis_index("c")
    sid = lax.axis_index("s")
    wid = sid * NC + cid
    cps = pltpu.async_copy(src4_hbm.at[wid], src_v, gsems[0])
    cpd = pltpu.async_copy(dst4_hbm.at[wid], dst_v, gsems[1])
    cph = pltpu.async_copy(h_hbm.at[pl.ds(sid * SPT, SPT)], h_v, gsems[2])
    cp0 = pltpu.async_copy(degp_hbm.at[0, sid], p0_v, gsems[3])
    cp1 = pltpu.async_copy(degp_hbm.at[1, sid], p1_v, ssems[0])
    _fill_rows(zero_v, SPT, 0.0)
    pltpu.sync_copy(zero_v, acc_sh.at[pl.ds(sid * SPT, SPT)])
    cph.wait()
    cp0.wait()
    cp1.wait()

    def srow(i, carry):
        iv = jnp.full((16,), i, jnp.int32)
        p = (plsc.load_gather(p0_v, [iv]) + plsc.load_gather(p1_v, [iv])
             + 1.0)
        y = _rsqrt16(p)
        dis_v[i, :] = y
        g_v[i, :] = h_v[i, :] * y
        return carry

    lax.fori_loop(0, SPT, srow, 0, unroll=4)
    pltpu.sync_copy(g_v, g_sh.at[pl.ds(sid * SPT, SPT)])

    @pl.when(cid == 0)
    def _():
        pltpu.sync_copy(g_v, g_out.at[pl.ds(sid * SPT, SPT)])
        pltpu.sync_copy(dis_v, dis_out.at[pl.ds(sid * SPT, SPT)])

    cps.wait()
    cpd.wait()
    plsc.subcore_barrier()

    for b in range(NBUF):
        pltpu.async_copy(g_sh.at[src_v.at[b]], bufs[b], gsems[b])

    def body(i, carry):
        for b in range(NBUF):
            j = NBUF * i + b
            pltpu.make_async_copy(g_sh.at[src_v.at[j]], bufs[b],
                                  gsems[b]).wait()
            pltpu.async_copy(bufs[b], acc_sh.at[dst_v.at[j]], ssems[b],
                             add=True)
        for b in range(NBUF):
            j = NBUF * i + b
            pltpu.make_async_copy(bufs[b], acc_sh.at[dst_v.at[j]],
                                  ssems[b]).wait()
            pltpu.async_copy(g_sh.at[src_v.at[j + NBUF]], bufs[b], gsems[b])
        return carry

    lax.fori_loop(0, NCH // NBUF - 1, body, 0)
    for b in range(NBUF):
        j = NCH - NBUF + b
        pltpu.make_async_copy(g_sh.at[src_v.at[j]], bufs[b], gsems[b]).wait()
        pltpu.async_copy(bufs[b], acc_sh.at[dst_v.at[j]], ssems[b], add=True)
    for b in range(NBUF):
        j = NCH - NBUF + b
        pltpu.make_async_copy(bufs[b], acc_sh.at[dst_v.at[j]], ssems[b]).wait()
    plsc.subcore_barrier()
    pltpu.sync_copy(acc_sh.at[pl.ds(sid * SPT, SPT)],
                    ap_out.at[cid, pl.ds(sid * SPT, SPT)])


@functools.partial(
    pl.kernel,
    out_type=jax.ShapeDtypeStruct((NC, APAD, H), jnp.float32),
    mesh=_mesh,
    scratch_types=[
        pltpu.VMEM((NCH, CHUNK), jnp.int32),
        pltpu.VMEM((NCH, CHUNK), jnp.int32),
        pltpu.VMEM((SPT, H), jnp.float32),
        pltpu.VMEM((SPT, H), jnp.float32),
        [pltpu.VMEM((CHUNK, H), jnp.float32)] * NBUF,
        pltpu.VMEM_SHARED((N, H), jnp.float32),
        pltpu.VMEM_SHARED((N, H), jnp.float32),
        [pltpu.SemaphoreType.DMA] * NBUF,
        [pltpu.SemaphoreType.DMA] * NBUF,
    ],
    compiler_params=_sc_params,
)
def _agg_kernel(g_hbm, src4_hbm, dst4_hbm, out_hbm,
                src_v, dst_v, zero_v, g_v, bufs, g_sh, acc_sh, gsems, ssems):
    cid = lax.axis_index("c")
    sid = lax.axis_index("s")
    wid = sid * NC + cid
    cps = pltpu.async_copy(src4_hbm.at[wid], src_v, gsems[0])
    cpd = pltpu.async_copy(dst4_hbm.at[wid], dst_v, gsems[1])
    cpg = pltpu.async_copy(g_hbm.at[pl.ds(sid * SPT, SPT)], g_v, gsems[2])
    _fill_rows(zero_v, SPT, 0.0)
    pltpu.sync_copy(zero_v, acc_sh.at[pl.ds(sid * SPT, SPT)])
    cpg.wait()
    pltpu.sync_copy(g_v, g_sh.at[pl.ds(sid * SPT, SPT)])
    cps.wait()
    cpd.wait()
    plsc.subcore_barrier()

    for b in range(NBUF):
        pltpu.async_copy(g_sh.at[src_v.at[b]], bufs[b], gsems[b])

    def body(i, carry):
        for b in range(NBUF):
            j = NBUF * i + b
            pltpu.make_async_copy(g_sh.at[src_v.at[j]], bufs[b],
                                  gsems[b]).wait()
            pltpu.async_copy(bufs[b], acc_sh.at[dst_v.at[j]], ssems[b],
                             add=True)
        for b in range(NBUF):
            j = NBUF * i + b
            pltpu.make_async_copy(bufs[b], acc_sh.at[dst_v.at[j]],
                                  ssems[b]).wait()
            pltpu.async_copy(g_sh.at[src_v.at[j + NBUF]], bufs[b], gsems[b])
        return carry

    lax.fori_loop(0, NCH // NBUF - 1, body, 0)
    for b in range(NBUF):
        j = NCH - NBUF + b
        pltpu.make_async_copy(g_sh.at[src_v.at[j]], bufs[b], gsems[b]).wait()
        pltpu.async_copy(bufs[b], acc_sh.at[dst_v.at[j]], ssems[b], add=True)
    for b in range(NBUF):
        j = NCH - NBUF + b
        pltpu.make_async_copy(bufs[b], acc_sh.at[dst_v.at[j]], ssems[b]).wait()
    plsc.subcore_barrier()
    pltpu.sync_copy(acc_sh.at[pl.ds(sid * SPT, SPT)],
                    out_hbm.at[cid, pl.ds(sid * SPT, SPT)])


def _matmul1_body(x_ref, w1_ref, h_ref):
    h_ref[...] = jnp.dot(x_ref[...], w1_ref[...],
                         preferred_element_type=jnp.float32)


def _dense2_body(gp_ref, ap_ref, disp_ref, b1t_ref, w2b_ref, outp_ref):
    dis = disp_ref[...]
    ap = ap_ref[...]
    z = jnp.maximum(
        dis * (gp_ref[...] + ap[0:PR] + ap[PRC:PRC + PR]) + b1t_ref[...], 0.0)
    outp_ref[...] = jnp.dot(z, w2b_ref[...],
                            preferred_element_type=jnp.float32) * dis


def _dense3_body(gp_ref, ap_ref, disp_ref, b2t_ref, outp_ref):
    ap = ap_ref[...]
    outp_ref[...] = (disp_ref[...] * (gp_ref[...] + ap[0:PR] + ap[PRC:PRC + PR])
                     + b2t_ref[...])


def kernel(x, edge_index, W1, b1, W2, b2):
    src4 = edge_index[0].reshape(NW, NCH, CHUNK)
    dst4 = edge_index[1].reshape(NW, NCH, CHUNK)
    w2b = jnp.kron(jnp.eye(8, dtype=jnp.float32), W2)
    b1t = jnp.tile(b1, 8).reshape(1, 128)
    b2t = jnp.tile(b2, 8).reshape(1, 128)
    pstruct = jax.ShapeDtypeStruct((PR, 128), jnp.float32)

    degp = _deg_kernel(dst4)

    h1 = pl.pallas_call(
        _matmul1_body,
        out_shape=jax.ShapeDtypeStruct((N, H), jnp.float32),
    )(x, W1)

    a1p, g1, dis = _scale_agg_kernel(h1, degp, src4, dst4)
    g1p = g1.reshape(PR, 128)
    disp = dis.reshape(PR, 128)
    g2p = pl.pallas_call(
        _dense2_body, out_shape=pstruct)(
            g1p, a1p.reshape(NC * PRC, 128), disp, b1t, w2b)

    a2p = _agg_kernel(g2p.reshape(N, H), src4, dst4)
    outp = pl.pallas_call(
        _dense3_body, out_shape=pstruct)(
            g2p, a2p.reshape(NC * PRC, 128), disp, b2t)
    return outp.reshape(N, C)
